# Initial kernel scaffold; baseline (speedup 1.0000x reference)
import jax
import jax.numpy as jnp
from jax.experimental import pallas as pl


def kernel(y, target, x, edge_index, W_msg, b_msg):
    raise NotImplementedError("write your pallas kernel here")


# speedup vs baseline: 4.0611x; 4.0611x over previous
import functools

import jax
import jax.numpy as jnp
from jax import lax
from jax.experimental import pallas as pl
from jax.experimental.pallas import tpu as pltpu
from jax.experimental.pallas import tpu_sc as plsc

_REG_WEIGHT = 0.01
_D = 128
_L = 16
_B = 80


def _tc_body(y_ref, t_ref, x_ref, wt_ref, wb_ref, b_ref, u_ref, v_ref, base_ref):
    x = x_ref[...]
    u_ref[...] = jnp.dot(x, wt_ref[...], precision=jax.lax.Precision.HIGHEST,
                         preferred_element_type=jnp.float32) + b_ref[...]
    v_ref[...] = jnp.dot(x, wb_ref[...], precision=jax.lax.Precision.HIGHEST,
                         preferred_element_type=jnp.float32)
    base_ref[...] = jnp.sum(jnp.abs(y_ref[...] - t_ref[...])).reshape(1, 1)


def _tc_stage(y2, t2, x, wt, wb, b2):
    n_nodes = x.shape[0]
    return pl.pallas_call(
        _tc_body,
        out_shape=[
            jax.ShapeDtypeStruct((n_nodes, _D), jnp.float32),
            jax.ShapeDtypeStruct((n_nodes, _D), jnp.float32),
            jax.ShapeDtypeStruct((1, 1), jnp.float32),
        ],
    )(y2, t2, x, wt, wb, b2)


def _make_sc_stage(n_edges):
    info = plsc.get_sparse_core_info()
    nc, ns = info.num_cores, info.num_subcores
    nw = nc * ns
    epw = n_edges // nw
    nblk = epw // _B
    assert epw * nw == n_edges and nblk * _B == epw

    mesh = plsc.VectorSubcoreMesh(core_axis_name="c", subcore_axis_name="s")

    @functools.partial(
        pl.kernel,
        mesh=mesh,
        out_type=jax.ShapeDtypeStruct((nw, _L), jnp.float32),
        scratch_types=[
            pltpu.VMEM((_B,), jnp.int32),
            pltpu.VMEM((_B,), jnp.int32),
            pltpu.VMEM((_B, _D), jnp.float32),
            pltpu.VMEM((_B, _D), jnp.float32),
            pltpu.VMEM((_L,), jnp.float32),
            pltpu.SemaphoreType.DMA,
            pltpu.SemaphoreType.DMA,
        ],
    )
    def sc_edge(u_hbm, v_hbm, src_hbm, dst_hbm, out_hbm,
                idxs_v, idxd_v, bufu, bufv, accv, sem_u, sem_v):
        wid = lax.axis_index("s") * nc + lax.axis_index("c")
        base = wid * epw

        def blk(g, acc):
            eb = base + g * _B
            pltpu.sync_copy(src_hbm.at[pl.ds(eb, _B)], idxs_v)
            pltpu.sync_copy(dst_hbm.at[pl.ds(eb, _B)], idxd_v)
            cu = pltpu.async_copy(u_hbm.at[idxs_v], bufu, sem_u)
            cv = pltpu.async_copy(v_hbm.at[idxd_v], bufv, sem_v)
            cu.wait()
            cv.wait()

            def row(j, a):
                def chunk(c, a2):
                    uu = bufu[j, pl.ds(c * _L, _L)]
                    vv = bufv[j, pl.ds(c * _L, _L)]
                    return a2 + jnp.abs(uu + vv)
                return lax.fori_loop(0, _D // _L, chunk, a)

            return lax.fori_loop(0, _B, row, acc)

        acc = lax.fori_loop(0, nblk, blk, jnp.zeros((_L,), jnp.float32))
        accv[...] = acc
        pltpu.sync_copy(accv, out_hbm.at[wid])

    return sc_edge


def kernel(y, target, x, edge_index, W_msg, b_msg):
    n_nodes = x.shape[0]
    n_edges = edge_index.shape[1]
    ei = edge_index.astype(jnp.int32)
    src, dst = ei[0], ei[1]
    wt, wb = W_msg[:_D], W_msg[_D:]
    y2 = y.reshape(80, n_nodes // 80)
    t2 = target.reshape(80, n_nodes // 80)
    u, v, base_sum = _tc_stage(y2, t2, x, wt, wb, b_msg.reshape(1, _D))
    parts = _make_sc_stage(n_edges)(u, v, src, dst)
    base_loss = base_sum[0, 0] / n_nodes
    l1_reg = jnp.sum(parts) / n_edges
    total_loss = base_loss + _REG_WEIGHT * l1_reg
    return (total_loss, base_loss, l1_reg)


# speedup vs baseline: 8.5408x; 2.1031x over previous
import functools

import jax
import jax.numpy as jnp
from jax import lax
from jax.experimental import pallas as pl
from jax.experimental.pallas import tpu as pltpu
from jax.experimental.pallas import tpu_sc as plsc

_REG_WEIGHT = 0.01
_D = 128
_L = 16
_B = 80


def _tc_body(y_ref, t_ref, x_ref, wt_ref, wb_ref, b_ref, u_ref, v_ref, base_ref):
    x = x_ref[...]
    u_ref[...] = jnp.dot(x, wt_ref[...], precision=jax.lax.Precision.HIGHEST,
                         preferred_element_type=jnp.float32) + b_ref[...]
    v_ref[...] = jnp.dot(x, wb_ref[...], precision=jax.lax.Precision.HIGHEST,
                         preferred_element_type=jnp.float32)
    base_ref[...] = jnp.sum(jnp.abs(y_ref[...] - t_ref[...])).reshape(1, 1)


def _tc_stage(y2, t2, x, wt, wb, b2):
    n_nodes = x.shape[0]
    return pl.pallas_call(
        _tc_body,
        out_shape=[
            jax.ShapeDtypeStruct((n_nodes, _D), jnp.float32),
            jax.ShapeDtypeStruct((n_nodes, _D), jnp.float32),
            jax.ShapeDtypeStruct((1, 1), jnp.float32),
        ],
    )(y2, t2, x, wt, wb, b2)


def _make_sc_stage(n_edges):
    info = plsc.get_sparse_core_info()
    nc, ns = info.num_cores, info.num_subcores
    nw = nc * ns
    epw = n_edges // nw
    nblk = epw // _B
    assert epw * nw == n_edges and nblk * _B == epw and nblk % 2 == 1

    mesh = plsc.VectorSubcoreMesh(core_axis_name="c", subcore_axis_name="s")

    @functools.partial(
        pl.kernel,
        mesh=mesh,
        compiler_params=pltpu.CompilerParams(needs_layout_passes=False),
        out_type=jax.ShapeDtypeStruct((nw, _L), jnp.float32),
        scratch_types=[
            pltpu.VMEM((epw,), jnp.int32),
            pltpu.VMEM((epw,), jnp.int32),
            pltpu.VMEM((2, _B, _D), jnp.float32),
            pltpu.VMEM((2, _B, _D), jnp.float32),
            pltpu.VMEM((_L,), jnp.float32),
            pltpu.SemaphoreType.DMA,
            pltpu.SemaphoreType.DMA,
            pltpu.SemaphoreType.DMA,
            pltpu.SemaphoreType.DMA,
        ],
    )
    def sc_edge(u_hbm, v_hbm, src_hbm, dst_hbm, out_hbm,
                idxs_v, idxd_v, bufu, bufv, accv, su0, sv0, su1, sv1):
        wid = lax.axis_index("s") * nc + lax.axis_index("c")
        base = wid * epw

        pltpu.sync_copy(src_hbm.at[pl.ds(base, epw)], idxs_v)
        pltpu.sync_copy(dst_hbm.at[pl.ds(base, epw)], idxd_v)

        sems = ((su0, sv0), (su1, sv1))

        def start(g, slot):
            su, sv = sems[slot]
            pltpu.async_copy(u_hbm.at[idxs_v.at[pl.ds(g * _B, _B)]],
                             bufu.at[slot], su)
            pltpu.async_copy(v_hbm.at[idxd_v.at[pl.ds(g * _B, _B)]],
                             bufv.at[slot], sv)

        def wait(g, slot):
            su, sv = sems[slot]
            pltpu.make_async_copy(u_hbm.at[idxs_v.at[pl.ds(g * _B, _B)]],
                                  bufu.at[slot], su).wait()
            pltpu.make_async_copy(v_hbm.at[idxd_v.at[pl.ds(g * _B, _B)]],
                                  bufv.at[slot], sv).wait()

        def consume(slot, acc):
            bu, bv = bufu.at[slot], bufv.at[slot]

            def row(j, a):
                for c in range(_D // _L):
                    cs = pl.ds(c * _L, _L)
                    a = a + jnp.abs(bu[j, cs] + bv[j, cs])
                return a

            return lax.fori_loop(0, _B, row, acc)

        start(0, 0)

        def pair(g2, acc):
            g0 = 2 * g2
            start(g0 + 1, 1)
            wait(g0, 0)
            acc = consume(0, acc)
            start(g0 + 2, 0)
            wait(g0 + 1, 1)
            return consume(1, acc)

        acc = lax.fori_loop(0, nblk // 2, pair, jnp.zeros((_L,), jnp.float32))
        wait(nblk - 1, 0)
        acc = consume(0, acc)

        accv[...] = acc
        pltpu.sync_copy(accv, out_hbm.at[wid])

    return sc_edge


def kernel(y, target, x, edge_index, W_msg, b_msg):
    n_nodes = x.shape[0]
    n_edges = edge_index.shape[1]
    ei = edge_index.astype(jnp.int32)
    src, dst = ei[0], ei[1]
    wt, wb = W_msg[:_D], W_msg[_D:]
    y2 = y.reshape(80, n_nodes // 80)
    t2 = target.reshape(80, n_nodes // 80)
    u, v, base_sum = _tc_stage(y2, t2, x, wt, wb, b_msg.reshape(1, _D))
    parts = _make_sc_stage(n_edges)(u, v, src, dst)
    base_loss = base_sum[0, 0] / n_nodes
    l1_reg = jnp.sum(parts) / n_edges
    total_loss = base_loss + _REG_WEIGHT * l1_reg
    return (total_loss, base_loss, l1_reg)
